# baseline (speedup 1.0000x reference)
import jax
import jax.numpy as jnp
from jax import lax
from jax.experimental import pallas as pl
from jax.experimental.pallas import tpu as pltpu
from jax.experimental.pallas import tpu_sc as plsc

SEQ = 200
BATCH = 4096
EMB = 64
ROW = 128
LANES = 16
NUM_CORES = 2
NUM_SUBCORES = 16
NUM_WORKERS = NUM_CORES * NUM_SUBCORES
NB = BATCH // NUM_WORKERS
NBUF = 4
ECHUNKS = EMB // LANES
IU = 4


def _body(table_hbm, x_hbm, out_hbm, idx_v, rows_v, acc_v,
          sem0, sem1, sem2, sem3):
    wid = lax.axis_index("s") * NUM_CORES + lax.axis_index("c")
    base = wid * NB
    sems = [sem0, sem1, sem2, sem3]

    pltpu.sync_copy(x_hbm.at[pl.ds(0, SEQ), pl.ds(base, NB)], idx_v)

    zero = jnp.zeros((LANES,), jnp.float32)

    def zero_body(i, carry):
        for j in range(ECHUNKS):
            acc_v[i, pl.ds(j * LANES, LANES)] = zero
        return carry

    lax.fori_loop(0, NB, zero_body, 0)

    def fire(s, b):
        pltpu.async_copy(table_hbm.at[idx_v.at[s]], rows_v.at[b], sems[b])

    def drain(b):
        pltpu.make_async_copy(
            table_hbm.at[idx_v.at[0]], rows_v.at[b], sems[b]).wait()

    for b in range(NBUF):
        fire(b, b)

    def outer(g, carry):
        for b in range(NBUF):
            s = g * NBUF + b
            drain(b)

            def add_body(t, c):
                for u in range(IU):
                    i = t * IU + u
                    for j in range(ECHUNKS):
                        plsc.addupdate(
                            acc_v.at[i, pl.ds(j * LANES, LANES)],
                            rows_v[b, i, pl.ds(j * LANES, LANES)])
                return c

            lax.fori_loop(0, NB // IU, add_body, 0)

            @pl.when(s + NBUF < SEQ)
            def _():
                fire(s + NBUF, b)
        return carry

    lax.fori_loop(0, SEQ // NBUF, outer, 0)

    inv = jnp.full((LANES,), 1.0 / SEQ, dtype=jnp.float32)

    def scale_body(i, carry):
        for j in range(ECHUNKS):
            sl = pl.ds(j * LANES, LANES)
            acc_v[i, sl] = acc_v[i, sl] * inv
        return carry

    lax.fori_loop(0, NB, scale_body, 0)
    pltpu.sync_copy(acc_v, out_hbm.at[pl.ds(base, NB)])


@jax.jit
def _fast_text(table, x):
    mesh = plsc.VectorSubcoreMesh(
        core_axis_name="c", subcore_axis_name="s",
        num_cores=NUM_CORES, num_subcores=NUM_SUBCORES)
    padded = jnp.concatenate([table, table], axis=1)
    grid_kernel = pl.kernel(
        _body,
        out_type=jax.ShapeDtypeStruct((BATCH, EMB), jnp.float32),
        mesh=mesh,
        compiler_params=pltpu.CompilerParams(use_tc_tiling_on_sc=True),
        scratch_types=[
            pltpu.VMEM((SEQ, NB), jnp.int32),
            pltpu.VMEM((NBUF, NB, ROW), jnp.float32),
            pltpu.VMEM((NB, EMB), jnp.float32),
            pltpu.SemaphoreType.DMA,
            pltpu.SemaphoreType.DMA,
            pltpu.SemaphoreType.DMA,
            pltpu.SemaphoreType.DMA,
        ],
    )
    return grid_kernel(padded, x)


def kernel(x, table):
    return _fast_text(table, x.astype(jnp.int32))


# speedup vs baseline: 1.5051x; 1.5051x over previous
import jax
import jax.numpy as jnp
from jax import lax
from jax.experimental import pallas as pl
from jax.experimental.pallas import tpu as pltpu
from jax.experimental.pallas import tpu_sc as plsc

SEQ = 200
BATCH = 4096
EMB = 64
LANES = 16
NUM_CORES = 2
NUM_SUBCORES = 16
NUM_WORKERS = NUM_CORES * NUM_SUBCORES
NB = BATCH // NUM_WORKERS
NBUF = 2
ECHUNKS = EMB // LANES
IU = 4


def _body(table_hbm, x_hbm, out_hbm, idx_v, rows_v, acc_v, sem0, sem1):
    wid = lax.axis_index("s") * NUM_CORES + lax.axis_index("c")
    base = wid * NB
    sems = [sem0, sem1]

    pltpu.sync_copy(x_hbm.at[pl.ds(0, SEQ), pl.ds(base, NB)], idx_v)

    zero = jnp.zeros((LANES,), jnp.float32)

    def zero_body(i, carry):
        for j in range(ECHUNKS):
            acc_v[i, pl.ds(j * LANES, LANES)] = zero
        return carry

    lax.fori_loop(0, NB, zero_body, 0)

    def fire(s, b):
        def grp(t, c):
            i0 = t * LANES
            idx16 = idx_v[s, pl.ds(i0, LANES)]
            for u in range(LANES):
                pltpu.async_copy(
                    table_hbm.at[pl.ds(idx16[u], 1)],
                    rows_v.at[b, pl.ds(i0 + u, 1)], sems[b])
            return c

        lax.fori_loop(0, NB // LANES, grp, 0)

    def drain(b):
        pltpu.make_async_copy(
            table_hbm.at[pl.ds(0, NB)], rows_v.at[b], sems[b]).wait()

    for b in range(NBUF):
        fire(b, b)

    def outer(g, carry):
        for b in range(NBUF):
            s = g * NBUF + b
            drain(b)

            def add_body(t, c):
                for u in range(IU):
                    i = t * IU + u
                    for j in range(ECHUNKS):
                        plsc.addupdate(
                            acc_v.at[i, pl.ds(j * LANES, LANES)],
                            rows_v[b, i, pl.ds(j * LANES, LANES)])
                return c

            lax.fori_loop(0, NB // IU, add_body, 0)

            @pl.when(s + NBUF < SEQ)
            def _():
                fire(s + NBUF, b)
        return carry

    lax.fori_loop(0, SEQ // NBUF, outer, 0)

    inv = jnp.full((LANES,), 1.0 / SEQ, dtype=jnp.float32)

    def scale_body(i, carry):
        for j in range(ECHUNKS):
            sl = pl.ds(j * LANES, LANES)
            acc_v[i, sl] = acc_v[i, sl] * inv
        return carry

    lax.fori_loop(0, NB, scale_body, 0)
    pltpu.sync_copy(acc_v, out_hbm.at[pl.ds(base, NB)])


@jax.jit
def _fast_text(table, x):
    mesh = plsc.VectorSubcoreMesh(
        core_axis_name="c", subcore_axis_name="s",
        num_cores=NUM_CORES, num_subcores=NUM_SUBCORES)
    grid_kernel = pl.kernel(
        _body,
        out_type=jax.ShapeDtypeStruct((BATCH, EMB), jnp.float32),
        mesh=mesh,
        compiler_params=pltpu.CompilerParams(use_tc_tiling_on_sc=True),
        scratch_types=[
            pltpu.VMEM((SEQ, NB), jnp.int32),
            pltpu.VMEM((NBUF, NB, EMB), jnp.float32),
            pltpu.VMEM((NB, EMB), jnp.float32),
            pltpu.SemaphoreType.DMA,
            pltpu.SemaphoreType.DMA,
        ],
    )
    return grid_kernel(table, x)


def kernel(x, table):
    return _fast_text(table, x.astype(jnp.int32))


# speedup vs baseline: 1.5248x; 1.0131x over previous
import jax
import jax.numpy as jnp
from jax import lax
from jax.experimental import pallas as pl
from jax.experimental.pallas import tpu as pltpu
from jax.experimental.pallas import tpu_sc as plsc

SEQ = 200
BATCH = 4096
EMB = 64
LANES = 16
NUM_CORES = 2
NUM_SUBCORES = 16
NUM_WORKERS = NUM_CORES * NUM_SUBCORES
NB = BATCH // NUM_WORKERS
NBUF = 2
ECHUNKS = EMB // LANES
IU = 4


def _body(table_hbm, x_hbm, out_hbm, idx_v, rows_v, acc_v, sem0, sem1):
    wid = lax.axis_index("s") * NUM_CORES + lax.axis_index("c")
    base = wid * NB
    sems = [sem0, sem1]

    pltpu.sync_copy(x_hbm.at[pl.ds(0, SEQ), pl.ds(base, NB)], idx_v)

    zero = jnp.zeros((LANES,), jnp.float32)

    def zero_body(i, carry):
        for j in range(ECHUNKS):
            acc_v[i, pl.ds(j * LANES, LANES)] = zero
        return carry

    lax.fori_loop(0, NB, zero_body, 0)

    def fire(s, b):
        def grp(t, c):
            i0 = t * LANES
            idx16 = idx_v[s, pl.ds(i0, LANES)]
            for u in range(LANES):
                pltpu.async_copy(
                    table_hbm.at[pl.ds(idx16[u], 1)],
                    rows_v.at[b, pl.ds(i0 + u, 1)], sems[b])
            return c

        lax.fori_loop(0, NB // LANES, grp, 0)

    def drain(b):
        pltpu.make_async_copy(
            table_hbm.at[pl.ds(0, NB)], rows_v.at[b], sems[b]).wait()

    for b in range(NBUF):
        fire(b, b)

    def outer(g, carry):
        for b in range(NBUF):
            s = g * NBUF + b
            drain(b)

            def add_body(t, c):
                i0 = t * LANES
                for u in range(LANES):
                    i = i0 + u
                    for j in range(ECHUNKS):
                        plsc.addupdate(
                            acc_v.at[i, pl.ds(j * LANES, LANES)],
                            rows_v[b, i, pl.ds(j * LANES, LANES)])

                @pl.when(s + NBUF < SEQ)
                def _():
                    idx16 = idx_v[s + NBUF, pl.ds(i0, LANES)]
                    for u in range(LANES):
                        pltpu.async_copy(
                            table_hbm.at[pl.ds(idx16[u], 1)],
                            rows_v.at[b, pl.ds(i0 + u, 1)], sems[b])
                return c

            lax.fori_loop(0, NB // LANES, add_body, 0)
        return carry

    lax.fori_loop(0, SEQ // NBUF, outer, 0)

    inv = jnp.full((LANES,), 1.0 / SEQ, dtype=jnp.float32)

    def scale_body(i, carry):
        for j in range(ECHUNKS):
            sl = pl.ds(j * LANES, LANES)
            acc_v[i, sl] = acc_v[i, sl] * inv
        return carry

    lax.fori_loop(0, NB, scale_body, 0)
    pltpu.sync_copy(acc_v, out_hbm.at[pl.ds(base, NB)])


@jax.jit
def _fast_text(table, x):
    mesh = plsc.VectorSubcoreMesh(
        core_axis_name="c", subcore_axis_name="s",
        num_cores=NUM_CORES, num_subcores=NUM_SUBCORES)
    grid_kernel = pl.kernel(
        _body,
        out_type=jax.ShapeDtypeStruct((BATCH, EMB), jnp.float32),
        mesh=mesh,
        compiler_params=pltpu.CompilerParams(use_tc_tiling_on_sc=True),
        scratch_types=[
            pltpu.VMEM((SEQ, NB), jnp.int32),
            pltpu.VMEM((NBUF, NB, EMB), jnp.float32),
            pltpu.VMEM((NB, EMB), jnp.float32),
            pltpu.SemaphoreType.DMA,
            pltpu.SemaphoreType.DMA,
        ],
    )
    return grid_kernel(table, x)


def kernel(x, table):
    return _fast_text(table, x.astype(jnp.int32))


# speedup vs baseline: 1.5253x; 1.0003x over previous
import jax
import jax.numpy as jnp
from jax import lax
from jax.experimental import pallas as pl
from jax.experimental.pallas import tpu as pltpu
from jax.experimental.pallas import tpu_sc as plsc

SEQ = 200
BATCH = 4096
EMB = 64
LANES = 16
NUM_CORES = 2
NUM_SUBCORES = 16
NUM_WORKERS = NUM_CORES * NUM_SUBCORES
NB = BATCH // NUM_WORKERS
NBUF = 4
ECHUNKS = EMB // LANES
IU = 4


def _body(table_hbm, x_hbm, out_hbm, idx_v, rows_v, acc_v,
          sem0, sem1, sem2, sem3):
    wid = lax.axis_index("s") * NUM_CORES + lax.axis_index("c")
    base = wid * NB
    sems = [sem0, sem1, sem2, sem3]

    pltpu.sync_copy(x_hbm.at[pl.ds(0, SEQ), pl.ds(base, NB)], idx_v)

    zero = jnp.zeros((LANES,), jnp.float32)

    def zero_body(i, carry):
        for j in range(ECHUNKS):
            acc_v[i, pl.ds(j * LANES, LANES)] = zero
        return carry

    lax.fori_loop(0, NB, zero_body, 0)

    def fire(s, b):
        def grp(t, c):
            i0 = t * LANES
            idx16 = idx_v[s, pl.ds(i0, LANES)]
            for u in range(LANES):
                pltpu.async_copy(
                    table_hbm.at[pl.ds(idx16[u], 1)],
                    rows_v.at[b, pl.ds(i0 + u, 1)], sems[b])
            return c

        lax.fori_loop(0, NB // LANES, grp, 0)

    def drain(b):
        pltpu.make_async_copy(
            table_hbm.at[pl.ds(0, NB)], rows_v.at[b], sems[b]).wait()

    for b in range(NBUF):
        fire(b, b)

    def outer(g, carry):
        for b in range(NBUF):
            s = g * NBUF + b
            drain(b)

            def add_body(t, c):
                i0 = t * LANES
                for u in range(LANES):
                    i = i0 + u
                    for j in range(ECHUNKS):
                        plsc.addupdate(
                            acc_v.at[i, pl.ds(j * LANES, LANES)],
                            rows_v[b, i, pl.ds(j * LANES, LANES)])

                @pl.when(s + NBUF < SEQ)
                def _():
                    idx16 = idx_v[s + NBUF, pl.ds(i0, LANES)]
                    for u in range(LANES):
                        pltpu.async_copy(
                            table_hbm.at[pl.ds(idx16[u], 1)],
                            rows_v.at[b, pl.ds(i0 + u, 1)], sems[b])
                return c

            lax.fori_loop(0, NB // LANES, add_body, 0)
        return carry

    lax.fori_loop(0, SEQ // NBUF, outer, 0)

    inv = jnp.full((LANES,), 1.0 / SEQ, dtype=jnp.float32)

    def scale_body(i, carry):
        for j in range(ECHUNKS):
            sl = pl.ds(j * LANES, LANES)
            acc_v[i, sl] = acc_v[i, sl] * inv
        return carry

    lax.fori_loop(0, NB, scale_body, 0)
    pltpu.sync_copy(acc_v, out_hbm.at[pl.ds(base, NB)])


@jax.jit
def _fast_text(table, x):
    mesh = plsc.VectorSubcoreMesh(
        core_axis_name="c", subcore_axis_name="s",
        num_cores=NUM_CORES, num_subcores=NUM_SUBCORES)
    grid_kernel = pl.kernel(
        _body,
        out_type=jax.ShapeDtypeStruct((BATCH, EMB), jnp.float32),
        mesh=mesh,
        compiler_params=pltpu.CompilerParams(use_tc_tiling_on_sc=True),
        scratch_types=[
            pltpu.VMEM((SEQ, NB), jnp.int32),
            pltpu.VMEM((NBUF, NB, EMB), jnp.float32),
            pltpu.VMEM((NB, EMB), jnp.float32),
            pltpu.SemaphoreType.DMA,
            pltpu.SemaphoreType.DMA,
            pltpu.SemaphoreType.DMA,
            pltpu.SemaphoreType.DMA,
        ],
    )
    return grid_kernel(table, x)


def kernel(x, table):
    return _fast_text(table, x.astype(jnp.int32))
